# baseline (speedup 1.0000x reference)
import functools
import math

import jax
import jax.numpy as jnp
from jax import lax
from jax.experimental import pallas as pl
from jax.experimental.pallas import tpu as pltpu
from jax.experimental.pallas import tpu_sc as plsc

H = 1024
B, T = 4, 8192
SCALE = math.sqrt(H)
NCOMBO = 729
POW3 = (1, 3, 9, 27, 81, 243)
NCOLS = 36

NC, NS = 2, 16
NW = NC * NS

S_SC = 10240
TB = 2048
CH = 32
NBUF = 3
RPW = S_SC // NW if S_SC else 0
NCH = RPW // CH if S_SC else 0
OFF_B = S_SC // TB
NB = (B * T) // TB


def _scalar_onehot(em_sm, sm_sm, pc_sm, rg_sm, b, n):
    parts = []
    for ref, width in ((em_sm, 4), (sm_sm, 4), (pc_sm, 5), (rg_sm, 5)):
        iw = lax.broadcasted_iota(jnp.int32, (n, width), 1)
        parts.append((iw == ref[b]).astype(jnp.float32))
    return parts


def _prep_body(em_sm, sm_sm, pc_sm, rg_sm, w_r,
               mix_r, fal_r, bre_r, pha_r, gli_r, vib_r,
               ct_r, cidx_r):
    b = pl.program_id(0)
    r = lax.broadcasted_iota(jnp.int32, (NCOMBO, 3), 0)
    i3 = lax.broadcasted_iota(jnp.int32, (NCOMBO, 3), 1)
    parts = [((r // POW3[k]) % 3 == i3).astype(jnp.float32) for k in range(6)]
    parts += _scalar_onehot(em_sm, sm_sm, pc_sm, rg_sm, b, NCOMBO)
    onehot = jnp.concatenate(parts, axis=1) * SCALE
    ct_r[0] = jnp.dot(onehot, w_r[...], preferred_element_type=jnp.float32)

    v = mix_r[0, 0, :]
    for t, ref in enumerate((fal_r, bre_r, pha_r, gli_r, vib_r)):
        v = v + ref[0, 0, :] * POW3[t + 1]
    cidx_r[0, 0, :] = v + b * NCOMBO


def _tc_body(em_sm, sm_sm, pc_sm, rg_sm, w_r,
             mix_r, fal_r, bre_r, pha_r, gli_r, vib_r, out_r):
    b = (pl.program_id(0) * TB + S_SC) // T
    i3 = lax.broadcasted_iota(jnp.int32, (TB, 3), 1)
    parts = [(ref[0, 0, :][:, None] == i3).astype(jnp.float32)
             for ref in (mix_r, fal_r, bre_r, pha_r, gli_r, vib_r)]
    parts += _scalar_onehot(em_sm, sm_sm, pc_sm, rg_sm, b, TB)
    onehot = jnp.concatenate(parts, axis=1) * SCALE
    out_r[...] = jnp.dot(onehot, w_r[...], preferred_element_type=jnp.float32)


def _sc_body(cidx_h, ct_h, out_h, cidx, rows, gs0, gs1, gs2, os0, os1, os2):
    wid = lax.axis_index("s") * NC + lax.axis_index("c")
    base = pl.multiple_of(wid * RPW, RPW)
    pltpu.sync_copy(cidx_h.at[pl.ds(base, RPW)], cidx)

    gsems = (gs0, gs1, gs2)
    osems = (os0, os1, os2)

    def _gather(i, buf):
        off = pl.multiple_of(i * CH, CH)
        return pltpu.make_async_copy(
            ct_h.at[cidx.at[pl.ds(off, CH)]], rows.at[buf], gsems[buf])

    def _out(i, buf):
        off = pl.multiple_of(base + i * CH, CH)
        return pltpu.make_async_copy(
            rows.at[buf], out_h.at[pl.ds(off, CH)], osems[buf])

    _gather(0, 0).start()
    _gather(1, 1).start()

    def _step(i):
        _gather(i, i % NBUF).wait()
        _out(i, i % NBUF).start()
        if i + 2 < NCH:
            if i >= 1:
                _out(i - 1, (i + 2) % NBUF).wait()
            _gather(i + 2, (i + 2) % NBUF).start()

    for i in range(NCH):
        _step(i)
    for i in (NCH - 2, NCH - 1):
        _out(i, i % NBUF).wait()


if S_SC:
    _sc_gather = functools.partial(
        pl.kernel,
        out_type=jax.ShapeDtypeStruct((S_SC, H), jnp.float32),
        mesh=plsc.VectorSubcoreMesh(core_axis_name="c", subcore_axis_name="s",
                                    num_cores=NC, num_subcores=NS),
        scratch_types=[
            pltpu.VMEM((RPW,), jnp.int32),
            pltpu.VMEM((NBUF, CH, H), jnp.float32),
            pltpu.SemaphoreType.DMA,
            pltpu.SemaphoreType.DMA,
            pltpu.SemaphoreType.DMA,
            pltpu.SemaphoreType.DMA,
            pltpu.SemaphoreType.DMA,
            pltpu.SemaphoreType.DMA,
        ],
    )(_sc_body)


def kernel(mix, falsetto, breathy, pharyngeal, glissando, vibrato,
           emotion, singing_method, pace, range_,
           mix_W, falsetto_W, breathy_W, pharyngeal_W, glissando_W, vibrato_W,
           emotion_W, singing_method_W, pace_W, range_W):
    wstack = jnp.concatenate(
        [mix_W, falsetto_W, breathy_W, pharyngeal_W, glissando_W, vibrato_W,
         emotion_W, singing_method_W, pace_W, range_W], axis=0)
    seq3d = [a.reshape(NB, 1, TB) for a in
             (mix, falsetto, breathy, pharyngeal, glissando, vibrato)]
    seqb = [a.reshape(B, 1, T) for a in
            (mix, falsetto, breathy, pharyngeal, glissando, vibrato)]
    scalars = (emotion, singing_method, pace, range_)

    smem = pl.BlockSpec(memory_space=pltpu.SMEM)
    wspec = pl.BlockSpec((NCOLS, H), lambda i: (0, 0))

    pieces = []
    if S_SC:
        ct, cidx = pl.pallas_call(
            _prep_body,
            grid=(B,),
            in_specs=[smem, smem, smem, smem, wspec]
                     + [pl.BlockSpec((1, 1, T), lambda b: (b, 0, 0))] * 6,
            out_specs=[pl.BlockSpec((1, NCOMBO, H), lambda b: (b, 0, 0)),
                       pl.BlockSpec((1, 1, T), lambda b: (b, 0, 0))],
            out_shape=[jax.ShapeDtypeStruct((B, NCOMBO, H), jnp.float32),
                       jax.ShapeDtypeStruct((B, 1, T), jnp.int32)],
        )(*scalars, wstack, *seqb)
        pieces.append(_sc_gather(cidx.reshape(B * T), ct.reshape(B * NCOMBO, H)))

    if S_SC < B * T:
        idx_spec = pl.BlockSpec((1, 1, TB), lambda i: (i + OFF_B, 0, 0))
        tc_out = pl.pallas_call(
            _tc_body,
            grid=(NB - OFF_B,),
            in_specs=[smem, smem, smem, smem, wspec] + [idx_spec] * 6,
            out_specs=pl.BlockSpec((TB, H), lambda i: (i, 0)),
            out_shape=jax.ShapeDtypeStruct((B * T - S_SC, H), jnp.float32),
        )(*scalars, wstack, *seq3d)
        pieces.append(tc_out)

    out = pieces[0] if len(pieces) == 1 else jnp.concatenate(pieces, axis=0)
    return out.reshape(B, T, H)


# speedup vs baseline: 1.3416x; 1.3416x over previous
import functools
import math

import jax
import jax.numpy as jnp
from jax import lax
from jax.experimental import pallas as pl
from jax.experimental.pallas import tpu as pltpu
from jax.experimental.pallas import tpu_sc as plsc

H = 1024
B, T = 4, 8192
SCALE = math.sqrt(H)
NCOMBO = 729
POW3 = (1, 3, 9, 27, 81, 243)
NCOLS = 36

NC, NS = 2, 16
NW = NC * NS

S_SC = 32768
TB = 2048
CH = 32
NBUF = 3
RPW = S_SC // NW if S_SC else 0
NCH = RPW // CH if S_SC else 0
OFF_B = S_SC // TB
NB = (B * T) // TB


def _scalar_onehot(em_sm, sm_sm, pc_sm, rg_sm, b, n):
    parts = []
    for ref, width in ((em_sm, 4), (sm_sm, 4), (pc_sm, 5), (rg_sm, 5)):
        iw = lax.broadcasted_iota(jnp.int32, (n, width), 1)
        parts.append((iw == ref[b]).astype(jnp.float32))
    return parts


def _prep_body(em_sm, sm_sm, pc_sm, rg_sm, w_r,
               mix_r, fal_r, bre_r, pha_r, gli_r, vib_r,
               ct_r, cidx_r):
    b = pl.program_id(0)
    r = lax.broadcasted_iota(jnp.int32, (NCOMBO, 3), 0)
    i3 = lax.broadcasted_iota(jnp.int32, (NCOMBO, 3), 1)
    parts = [((r // POW3[k]) % 3 == i3).astype(jnp.float32) for k in range(6)]
    parts += _scalar_onehot(em_sm, sm_sm, pc_sm, rg_sm, b, NCOMBO)
    onehot = jnp.concatenate(parts, axis=1) * SCALE
    ct_r[0] = jnp.dot(onehot, w_r[...], preferred_element_type=jnp.float32)

    v = mix_r[0, 0, :]
    for t, ref in enumerate((fal_r, bre_r, pha_r, gli_r, vib_r)):
        v = v + ref[0, 0, :] * POW3[t + 1]
    cidx_r[0, 0, :] = v + b * NCOMBO


def _tc_body(em_sm, sm_sm, pc_sm, rg_sm, w_r,
             mix_r, fal_r, bre_r, pha_r, gli_r, vib_r, out_r):
    b = (pl.program_id(0) * TB + S_SC) // T
    i3 = lax.broadcasted_iota(jnp.int32, (TB, 3), 1)
    parts = [(ref[0, 0, :][:, None] == i3).astype(jnp.float32)
             for ref in (mix_r, fal_r, bre_r, pha_r, gli_r, vib_r)]
    parts += _scalar_onehot(em_sm, sm_sm, pc_sm, rg_sm, b, TB)
    onehot = jnp.concatenate(parts, axis=1) * SCALE
    out_r[...] = jnp.dot(onehot, w_r[...], preferred_element_type=jnp.float32)


def _sc_body(cidx_h, ct_h, out_h, cidx, rows, gs0, gs1, gs2, os0, os1, os2):
    wid = lax.axis_index("s") * NC + lax.axis_index("c")
    base = pl.multiple_of(wid * RPW, RPW)
    pltpu.sync_copy(cidx_h.at[pl.ds(base, RPW)], cidx)

    gsems = (gs0, gs1, gs2)
    osems = (os0, os1, os2)

    def _gather(i, buf):
        off = pl.multiple_of(i * CH, CH)
        return pltpu.make_async_copy(
            ct_h.at[cidx.at[pl.ds(off, CH)]], rows.at[buf], gsems[buf])

    def _out(i, buf):
        off = pl.multiple_of(base + i * CH, CH)
        return pltpu.make_async_copy(
            rows.at[buf], out_h.at[pl.ds(off, CH)], osems[buf])

    _gather(0, 0).start()
    _gather(1, 1).start()

    def _step(i):
        _gather(i, i % NBUF).wait()
        _out(i, i % NBUF).start()
        if i + 2 < NCH:
            if i >= 1:
                _out(i - 1, (i + 2) % NBUF).wait()
            _gather(i + 2, (i + 2) % NBUF).start()

    for i in range(NCH):
        _step(i)
    for i in (NCH - 2, NCH - 1):
        _out(i, i % NBUF).wait()


if S_SC:
    _sc_gather = functools.partial(
        pl.kernel,
        out_type=jax.ShapeDtypeStruct((S_SC, H), jnp.float32),
        mesh=plsc.VectorSubcoreMesh(core_axis_name="c", subcore_axis_name="s",
                                    num_cores=NC, num_subcores=NS),
        scratch_types=[
            pltpu.VMEM((RPW,), jnp.int32),
            pltpu.VMEM((NBUF, CH, H), jnp.float32),
            pltpu.SemaphoreType.DMA,
            pltpu.SemaphoreType.DMA,
            pltpu.SemaphoreType.DMA,
            pltpu.SemaphoreType.DMA,
            pltpu.SemaphoreType.DMA,
            pltpu.SemaphoreType.DMA,
        ],
    )(_sc_body)


def kernel(mix, falsetto, breathy, pharyngeal, glissando, vibrato,
           emotion, singing_method, pace, range_,
           mix_W, falsetto_W, breathy_W, pharyngeal_W, glissando_W, vibrato_W,
           emotion_W, singing_method_W, pace_W, range_W):
    wstack = jnp.concatenate(
        [mix_W, falsetto_W, breathy_W, pharyngeal_W, glissando_W, vibrato_W,
         emotion_W, singing_method_W, pace_W, range_W], axis=0)
    seq3d = [a.reshape(NB, 1, TB) for a in
             (mix, falsetto, breathy, pharyngeal, glissando, vibrato)]
    seqb = [a.reshape(B, 1, T) for a in
            (mix, falsetto, breathy, pharyngeal, glissando, vibrato)]
    scalars = (emotion, singing_method, pace, range_)

    smem = pl.BlockSpec(memory_space=pltpu.SMEM)
    wspec = pl.BlockSpec((NCOLS, H), lambda i: (0, 0))

    pieces = []
    if S_SC:
        ct, cidx = pl.pallas_call(
            _prep_body,
            grid=(B,),
            in_specs=[smem, smem, smem, smem, wspec]
                     + [pl.BlockSpec((1, 1, T), lambda b: (b, 0, 0))] * 6,
            out_specs=[pl.BlockSpec((1, NCOMBO, H), lambda b: (b, 0, 0)),
                       pl.BlockSpec((1, 1, T), lambda b: (b, 0, 0))],
            out_shape=[jax.ShapeDtypeStruct((B, NCOMBO, H), jnp.float32),
                       jax.ShapeDtypeStruct((B, 1, T), jnp.int32)],
        )(*scalars, wstack, *seqb)
        pieces.append(_sc_gather(cidx.reshape(B * T), ct.reshape(B * NCOMBO, H)))

    if S_SC < B * T:
        idx_spec = pl.BlockSpec((1, 1, TB), lambda i: (i + OFF_B, 0, 0))
        tc_out = pl.pallas_call(
            _tc_body,
            grid=(NB - OFF_B,),
            in_specs=[smem, smem, smem, smem, wspec] + [idx_spec] * 6,
            out_specs=pl.BlockSpec((TB, H), lambda i: (i, 0)),
            out_shape=jax.ShapeDtypeStruct((B * T - S_SC, H), jnp.float32),
        )(*scalars, wstack, *seq3d)
        pieces.append(tc_out)

    out = pieces[0] if len(pieces) == 1 else jnp.concatenate(pieces, axis=0)
    return out.reshape(B, T, H)


# speedup vs baseline: 1.4355x; 1.0700x over previous
import functools
import math

import jax
import jax.numpy as jnp
from jax import lax
from jax.experimental import pallas as pl
from jax.experimental.pallas import tpu as pltpu
from jax.experimental.pallas import tpu_sc as plsc

H = 1024
B, T = 4, 8192
SCALE = math.sqrt(H)
NCOMBO = 729
NROWS = 736
POW3 = (1, 3, 9, 27, 81, 243)
NCOLS = 36

NC, NS = 2, 16
NW = NC * NS

S_SC = 32768
TB = 2048
CH = 32
NBUF = 3
RPW = S_SC // NW if S_SC else 0
NCH = RPW // CH if S_SC else 0
OFF_B = S_SC // TB
NB = (B * T) // TB


def _scalar_onehot(em_sm, sm_sm, pc_sm, rg_sm, b, n):
    parts = []
    for ref, width in ((em_sm, 4), (sm_sm, 4), (pc_sm, 5), (rg_sm, 5)):
        iw = lax.broadcasted_iota(jnp.int32, (n, width), 1)
        parts.append((iw == ref[b]).astype(jnp.float32))
    return parts


def _prep_body(em_sm, sm_sm, pc_sm, rg_sm, w_r,
               mix_r, fal_r, bre_r, pha_r, gli_r, vib_r,
               ct_r, cidx_r):
    b = pl.program_id(0)
    r = lax.broadcasted_iota(jnp.int32, (NROWS, 3), 0)
    i3 = lax.broadcasted_iota(jnp.int32, (NROWS, 3), 1)
    parts = [((r // POW3[k]) % 3 == i3).astype(jnp.float32) for k in range(6)]
    parts += _scalar_onehot(em_sm, sm_sm, pc_sm, rg_sm, b, NROWS)
    onehot = jnp.concatenate(parts, axis=1) * SCALE
    ct_r[0] = jnp.dot(onehot, w_r[...], preferred_element_type=jnp.float32)

    v = mix_r[0, 0, :]
    for t, ref in enumerate((fal_r, bre_r, pha_r, gli_r, vib_r)):
        v = v + ref[0, 0, :] * POW3[t + 1]
    cidx_r[0, 0, :] = v + b * NROWS


def _tc_body(em_sm, sm_sm, pc_sm, rg_sm, w_r,
             mix_r, fal_r, bre_r, pha_r, gli_r, vib_r, out_r):
    b = (pl.program_id(0) * TB + S_SC) // T
    i3 = lax.broadcasted_iota(jnp.int32, (TB, 3), 1)
    parts = [(ref[0, 0, :][:, None] == i3).astype(jnp.float32)
             for ref in (mix_r, fal_r, bre_r, pha_r, gli_r, vib_r)]
    parts += _scalar_onehot(em_sm, sm_sm, pc_sm, rg_sm, b, TB)
    onehot = jnp.concatenate(parts, axis=1) * SCALE
    out_r[...] = jnp.dot(onehot, w_r[...], preferred_element_type=jnp.float32)


def _sc_body(cidx_h, ct_h, out_h, cidx, rows, gs0, gs1, gs2, os0, os1, os2):
    wid = lax.axis_index("s") * NC + lax.axis_index("c")
    base = pl.multiple_of(wid * RPW, RPW)
    pltpu.sync_copy(cidx_h.at[pl.ds(base, RPW)], cidx)

    gsems = (gs0, gs1, gs2)
    osems = (os0, os1, os2)

    def _gather(i, buf):
        off = pl.multiple_of(i * CH, CH)
        return pltpu.make_async_copy(
            ct_h.at[cidx.at[pl.ds(off, CH)]], rows.at[buf], gsems[buf])

    def _out(i, buf):
        off = pl.multiple_of(base + i * CH, CH)
        return pltpu.make_async_copy(
            rows.at[buf], out_h.at[pl.ds(off, CH)], osems[buf])

    _gather(0, 0).start()
    _gather(1, 1).start()

    def _step(i):
        _gather(i, i % NBUF).wait()
        _out(i, i % NBUF).start()
        if i + 2 < NCH:
            if i >= 1:
                _out(i - 1, (i + 2) % NBUF).wait()
            _gather(i + 2, (i + 2) % NBUF).start()

    for i in range(NCH):
        _step(i)
    for i in (NCH - 2, NCH - 1):
        _out(i, i % NBUF).wait()


if S_SC:
    _sc_gather = functools.partial(
        pl.kernel,
        out_type=jax.ShapeDtypeStruct((S_SC, H), jnp.float32),
        mesh=plsc.VectorSubcoreMesh(core_axis_name="c", subcore_axis_name="s",
                                    num_cores=NC, num_subcores=NS),
        scratch_types=[
            pltpu.VMEM((RPW,), jnp.int32),
            pltpu.VMEM((NBUF, CH, H), jnp.float32),
            pltpu.SemaphoreType.DMA,
            pltpu.SemaphoreType.DMA,
            pltpu.SemaphoreType.DMA,
            pltpu.SemaphoreType.DMA,
            pltpu.SemaphoreType.DMA,
            pltpu.SemaphoreType.DMA,
        ],
    )(_sc_body)


def kernel(mix, falsetto, breathy, pharyngeal, glissando, vibrato,
           emotion, singing_method, pace, range_,
           mix_W, falsetto_W, breathy_W, pharyngeal_W, glissando_W, vibrato_W,
           emotion_W, singing_method_W, pace_W, range_W):
    wstack = jnp.concatenate(
        [mix_W, falsetto_W, breathy_W, pharyngeal_W, glissando_W, vibrato_W,
         emotion_W, singing_method_W, pace_W, range_W], axis=0)
    seq3d = [a.reshape(NB, 1, TB) for a in
             (mix, falsetto, breathy, pharyngeal, glissando, vibrato)]
    seqb = [a.reshape(B, 1, T) for a in
            (mix, falsetto, breathy, pharyngeal, glissando, vibrato)]
    scalars = (emotion, singing_method, pace, range_)

    smem = pl.BlockSpec(memory_space=pltpu.SMEM)
    wspec = pl.BlockSpec((NCOLS, H), lambda i: (0, 0))

    pieces = []
    if S_SC:
        ct, cidx = pl.pallas_call(
            _prep_body,
            grid=(B,),
            in_specs=[smem, smem, smem, smem, wspec]
                     + [pl.BlockSpec((1, 1, T), lambda b: (b, 0, 0))] * 6,
            out_specs=[pl.BlockSpec((1, NROWS, H), lambda b: (b, 0, 0)),
                       pl.BlockSpec((1, 1, T), lambda b: (b, 0, 0))],
            out_shape=[jax.ShapeDtypeStruct((B, NROWS, H), jnp.float32),
                       jax.ShapeDtypeStruct((B, 1, T), jnp.int32)],
        )(*scalars, wstack, *seqb)
        pieces.append(_sc_gather(cidx.reshape(B * T), ct.reshape(B * NROWS, H)))

    if S_SC < B * T:
        idx_spec = pl.BlockSpec((1, 1, TB), lambda i: (i + OFF_B, 0, 0))
        tc_out = pl.pallas_call(
            _tc_body,
            grid=(NB - OFF_B,),
            in_specs=[smem, smem, smem, smem, wspec] + [idx_spec] * 6,
            out_specs=pl.BlockSpec((TB, H), lambda i: (i, 0)),
            out_shape=jax.ShapeDtypeStruct((B * T - S_SC, H), jnp.float32),
        )(*scalars, wstack, *seq3d)
        pieces.append(tc_out)

    out = pieces[0] if len(pieces) == 1 else jnp.concatenate(pieces, axis=0)
    return out.reshape(B, T, H)


# speedup vs baseline: 1.4400x; 1.0031x over previous
import functools
import math

import jax
import jax.numpy as jnp
from jax import lax
from jax.experimental import pallas as pl
from jax.experimental.pallas import tpu as pltpu
from jax.experimental.pallas import tpu_sc as plsc

H = 1024
B, T = 4, 8192
SCALE = math.sqrt(H)
NCOMBO = 729
NROWS = 736
POW3 = (1, 3, 9, 27, 81, 243)
NCOLS = 36

NC, NS = 2, 16
NW = NC * NS

S_SC = 32768
TB = 2048
CH = 32
NBUF = 3
RPW = S_SC // NW if S_SC else 0
NCH = RPW // CH if S_SC else 0
OFF_B = S_SC // TB
NB = (B * T) // TB


def _scalar_onehot(em_sm, sm_sm, pc_sm, rg_sm, b, n):
    parts = []
    for ref, width in ((em_sm, 4), (sm_sm, 4), (pc_sm, 5), (rg_sm, 5)):
        iw = lax.broadcasted_iota(jnp.int32, (n, width), 1)
        parts.append((iw == ref[b]).astype(jnp.float32))
    return parts


def _prep_body(em_sm, sm_sm, pc_sm, rg_sm, w_r,
               mix_r, fal_r, bre_r, pha_r, gli_r, vib_r,
               ct_r, cidx_r):
    b = pl.program_id(0)
    r = lax.broadcasted_iota(jnp.int32, (NROWS, 3), 0)
    i3 = lax.broadcasted_iota(jnp.int32, (NROWS, 3), 1)
    parts = [((r // POW3[k]) % 3 == i3).astype(jnp.float32) for k in range(6)]
    parts += _scalar_onehot(em_sm, sm_sm, pc_sm, rg_sm, b, NROWS)
    onehot = jnp.concatenate(parts, axis=1) * SCALE
    ct_r[0] = jnp.dot(onehot, w_r[...], preferred_element_type=jnp.float32)

    v = mix_r[0, 0, :]
    for t, ref in enumerate((fal_r, bre_r, pha_r, gli_r, vib_r)):
        v = v + ref[0, 0, :] * POW3[t + 1]
    cidx_r[0, 0, :] = v + b * NROWS


def _tc_body(em_sm, sm_sm, pc_sm, rg_sm, w_r,
             mix_r, fal_r, bre_r, pha_r, gli_r, vib_r, out_r):
    b = (pl.program_id(0) * TB + S_SC) // T
    i3 = lax.broadcasted_iota(jnp.int32, (TB, 3), 1)
    parts = [(ref[0, 0, :][:, None] == i3).astype(jnp.float32)
             for ref in (mix_r, fal_r, bre_r, pha_r, gli_r, vib_r)]
    parts += _scalar_onehot(em_sm, sm_sm, pc_sm, rg_sm, b, TB)
    onehot = jnp.concatenate(parts, axis=1) * SCALE
    out_r[...] = jnp.dot(onehot, w_r[...], preferred_element_type=jnp.float32)


def _sc_body(cidx_h, ct_h, out_h, cidx, rows, gs0, gs1, gs2, os0, os1, os2):
    wid = lax.axis_index("s") * NC + lax.axis_index("c")
    base = pl.multiple_of(wid * RPW, RPW)
    pltpu.sync_copy(cidx_h.at[pl.ds(base, RPW)], cidx)

    gsems = (gs0, gs1, gs2)
    osems = (os0, os1, os2)

    def _gather(i, buf):
        off = pl.multiple_of(i * CH, CH)
        return pltpu.make_async_copy(
            ct_h.at[cidx.at[pl.ds(off, CH)]], rows.at[buf], gsems[buf])

    def _out(i, buf):
        off = pl.multiple_of(base + i * CH, CH)
        return pltpu.make_async_copy(
            rows.at[buf], out_h.at[pl.ds(off, CH)], osems[buf])

    _gather(0, 0).start()
    _gather(1, 1).start()

    def _step(i):
        _gather(i, i % NBUF).wait()
        _out(i, i % NBUF).start()
        if i + 2 < NCH:
            if i >= 1:
                _out(i - 1, (i + 2) % NBUF).wait()
            _gather(i + 2, (i + 2) % NBUF).start()

    for i in range(NCH):
        _step(i)
    for i in range(NCH - NBUF, NCH):
        _out(i, i % NBUF).wait()


def _make_sc_gather():
    return functools.partial(
        pl.kernel,
        out_type=jax.ShapeDtypeStruct((S_SC, H), jnp.float32),
        mesh=plsc.VectorSubcoreMesh(core_axis_name="c", subcore_axis_name="s",
                                    num_cores=NC, num_subcores=NS),
        scratch_types=[
            pltpu.VMEM((RPW,), jnp.int32),
            pltpu.VMEM((NBUF, CH, H), jnp.float32),
            pltpu.SemaphoreType.DMA,
            pltpu.SemaphoreType.DMA,
            pltpu.SemaphoreType.DMA,
            pltpu.SemaphoreType.DMA,
            pltpu.SemaphoreType.DMA,
            pltpu.SemaphoreType.DMA,
        ],
    )(_sc_body)


def kernel(mix, falsetto, breathy, pharyngeal, glissando, vibrato,
           emotion, singing_method, pace, range_,
           mix_W, falsetto_W, breathy_W, pharyngeal_W, glissando_W, vibrato_W,
           emotion_W, singing_method_W, pace_W, range_W):
    wstack = jnp.concatenate(
        [mix_W, falsetto_W, breathy_W, pharyngeal_W, glissando_W, vibrato_W,
         emotion_W, singing_method_W, pace_W, range_W], axis=0)
    seq3d = [a.reshape(NB, 1, TB) for a in
             (mix, falsetto, breathy, pharyngeal, glissando, vibrato)]
    seqb = [a.reshape(B, 1, T) for a in
            (mix, falsetto, breathy, pharyngeal, glissando, vibrato)]
    scalars = (emotion, singing_method, pace, range_)

    smem = pl.BlockSpec(memory_space=pltpu.SMEM)
    wspec = pl.BlockSpec((NCOLS, H), lambda i: (0, 0))

    pieces = []
    if S_SC:
        ct, cidx = pl.pallas_call(
            _prep_body,
            grid=(B,),
            in_specs=[smem, smem, smem, smem, wspec]
                     + [pl.BlockSpec((1, 1, T), lambda b: (b, 0, 0))] * 6,
            out_specs=[pl.BlockSpec((1, NROWS, H), lambda b: (b, 0, 0)),
                       pl.BlockSpec((1, 1, T), lambda b: (b, 0, 0))],
            out_shape=[jax.ShapeDtypeStruct((B, NROWS, H), jnp.float32),
                       jax.ShapeDtypeStruct((B, 1, T), jnp.int32)],
        )(*scalars, wstack, *seqb)
        pieces.append(_make_sc_gather()(cidx.reshape(B * T),
                                        ct.reshape(B * NROWS, H)))

    if S_SC < B * T:
        idx_spec = pl.BlockSpec((1, 1, TB), lambda i: (i + OFF_B, 0, 0))
        tc_out = pl.pallas_call(
            _tc_body,
            grid=(NB - OFF_B,),
            in_specs=[smem, smem, smem, smem, wspec] + [idx_spec] * 6,
            out_specs=pl.BlockSpec((TB, H), lambda i: (i, 0)),
            out_shape=jax.ShapeDtypeStruct((B * T - S_SC, H), jnp.float32),
        )(*scalars, wstack, *seq3d)
        pieces.append(tc_out)

    out = pieces[0] if len(pieces) == 1 else jnp.concatenate(pieces, axis=0)
    return out.reshape(B, T, H)


# speedup vs baseline: 1.4810x; 1.0285x over previous
import functools
import math

import jax
import jax.numpy as jnp
from jax import lax
from jax.experimental import pallas as pl
from jax.experimental.pallas import tpu as pltpu
from jax.experimental.pallas import tpu_sc as plsc

H = 1024
B, T = 4, 8192
SCALE = math.sqrt(H)
NCOMBO = 729
NROWS = 736
POW3 = (1, 3, 9, 27, 81, 243)
NCOLS = 36

TS, TL = 64, 128

NC, NS = 2, 16
NW = NC * NS

S_SC = 32768
TB = 2048
CH = 32
NBUF = 3
RPW = S_SC // NW if S_SC else 0
NCH = RPW // CH if S_SC else 0
OFF_B = S_SC // TB
NB = (B * T) // TB


def _scalar_onehot(em_sm, sm_sm, pc_sm, rg_sm, b, n):
    parts = []
    for ref, width in ((em_sm, 4), (sm_sm, 4), (pc_sm, 5), (rg_sm, 5)):
        iw = lax.broadcasted_iota(jnp.int32, (n, width), 1)
        parts.append((iw == ref[b]).astype(jnp.float32))
    return parts


def _prep_body(em_sm, sm_sm, pc_sm, rg_sm, w_r,
               mix_r, fal_r, bre_r, pha_r, gli_r, vib_r,
               ct_r, cidx_r):
    b = pl.program_id(0)
    r = lax.broadcasted_iota(jnp.int32, (NROWS, 3), 0)
    i3 = lax.broadcasted_iota(jnp.int32, (NROWS, 3), 1)
    parts = [((r // POW3[k]) % 3 == i3).astype(jnp.float32) for k in range(6)]
    parts += _scalar_onehot(em_sm, sm_sm, pc_sm, rg_sm, b, NROWS)
    onehot = jnp.concatenate(parts, axis=1) * SCALE
    ct_r[0] = jnp.dot(onehot, w_r[...], preferred_element_type=jnp.float32)

    v = mix_r[0]
    for t, ref in enumerate((fal_r, bre_r, pha_r, gli_r, vib_r)):
        v = v + ref[0] * POW3[t + 1]
    cidx_r[0] = v + b * NROWS


def _tc_body(em_sm, sm_sm, pc_sm, rg_sm, w_r,
             mix_r, fal_r, bre_r, pha_r, gli_r, vib_r, out_r):
    b = (pl.program_id(0) * TB + S_SC) // T
    i3 = lax.broadcasted_iota(jnp.int32, (TB, 3), 1)
    parts = [(ref[0, 0, :][:, None] == i3).astype(jnp.float32)
             for ref in (mix_r, fal_r, bre_r, pha_r, gli_r, vib_r)]
    parts += _scalar_onehot(em_sm, sm_sm, pc_sm, rg_sm, b, TB)
    onehot = jnp.concatenate(parts, axis=1) * SCALE
    out_r[...] = jnp.dot(onehot, w_r[...], preferred_element_type=jnp.float32)


def _sc_body(cidx_h, ct_h, out_h, cidx, rows, gs0, gs1, gs2, os0, os1, os2):
    wid = lax.axis_index("s") * NC + lax.axis_index("c")
    base = pl.multiple_of(wid * RPW, RPW)
    pltpu.sync_copy(cidx_h.at[pl.ds(base, RPW)], cidx)

    gsems = (gs0, gs1, gs2)
    osems = (os0, os1, os2)

    def _gather(i, buf):
        off = pl.multiple_of(i * CH, CH)
        return pltpu.make_async_copy(
            ct_h.at[cidx.at[pl.ds(off, CH)]], rows.at[buf], gsems[buf])

    def _out(i, buf):
        off = pl.multiple_of(base + i * CH, CH)
        return pltpu.make_async_copy(
            rows.at[buf], out_h.at[pl.ds(off, CH)], osems[buf])

    _gather(0, 0).start()
    _gather(1, 1).start()

    def _step(i):
        _gather(i, i % NBUF).wait()
        _out(i, i % NBUF).start()
        if i + 2 < NCH:
            if i >= 1:
                _out(i - 1, (i + 2) % NBUF).wait()
            _gather(i + 2, (i + 2) % NBUF).start()

    for i in range(NCH):
        _step(i)
    for i in range(NCH - NBUF, NCH):
        _out(i, i % NBUF).wait()


def _make_sc_gather():
    return functools.partial(
        pl.kernel,
        out_type=jax.ShapeDtypeStruct((S_SC, H), jnp.float32),
        mesh=plsc.VectorSubcoreMesh(core_axis_name="c", subcore_axis_name="s",
                                    num_cores=NC, num_subcores=NS),
        scratch_types=[
            pltpu.VMEM((RPW,), jnp.int32),
            pltpu.VMEM((NBUF, CH, H), jnp.float32),
            pltpu.SemaphoreType.DMA,
            pltpu.SemaphoreType.DMA,
            pltpu.SemaphoreType.DMA,
            pltpu.SemaphoreType.DMA,
            pltpu.SemaphoreType.DMA,
            pltpu.SemaphoreType.DMA,
        ],
    )(_sc_body)


def kernel(mix, falsetto, breathy, pharyngeal, glissando, vibrato,
           emotion, singing_method, pace, range_,
           mix_W, falsetto_W, breathy_W, pharyngeal_W, glissando_W, vibrato_W,
           emotion_W, singing_method_W, pace_W, range_W):
    wstack = jnp.concatenate(
        [mix_W, falsetto_W, breathy_W, pharyngeal_W, glissando_W, vibrato_W,
         emotion_W, singing_method_W, pace_W, range_W], axis=0)
    seq3d = [a.reshape(NB, 1, TB) for a in
             (mix, falsetto, breathy, pharyngeal, glissando, vibrato)]
    seqb = [a.reshape(B, TS, TL) for a in
            (mix, falsetto, breathy, pharyngeal, glissando, vibrato)]
    scalars = (emotion, singing_method, pace, range_)

    smem = pl.BlockSpec(memory_space=pltpu.SMEM)
    wspec = pl.BlockSpec((NCOLS, H), lambda i: (0, 0))

    pieces = []
    if S_SC:
        ct, cidx = pl.pallas_call(
            _prep_body,
            grid=(B,),
            in_specs=[smem, smem, smem, smem, wspec]
                     + [pl.BlockSpec((1, TS, TL), lambda b: (b, 0, 0))] * 6,
            out_specs=[pl.BlockSpec((1, NROWS, H), lambda b: (b, 0, 0)),
                       pl.BlockSpec((1, TS, TL), lambda b: (b, 0, 0))],
            out_shape=[jax.ShapeDtypeStruct((B, NROWS, H), jnp.float32),
                       jax.ShapeDtypeStruct((B, TS, TL), jnp.int32)],
        )(*scalars, wstack, *seqb)
        pieces.append(_make_sc_gather()(cidx.reshape(B * T),
                                        ct.reshape(B * NROWS, H)))

    if S_SC < B * T:
        idx_spec = pl.BlockSpec((1, 1, TB), lambda i: (i + OFF_B, 0, 0))
        tc_out = pl.pallas_call(
            _tc_body,
            grid=(NB - OFF_B,),
            in_specs=[smem, smem, smem, smem, wspec] + [idx_spec] * 6,
            out_specs=pl.BlockSpec((TB, H), lambda i: (i, 0)),
            out_shape=jax.ShapeDtypeStruct((B * T - S_SC, H), jnp.float32),
        )(*scalars, wstack, *seq3d)
        pieces.append(tc_out)

    out = pieces[0] if len(pieces) == 1 else jnp.concatenate(pieces, axis=0)
    return out.reshape(B, T, H)
